# baseline (speedup 1.0000x reference)
import functools

import jax
import jax.numpy as jnp
from jax import lax
from jax.experimental import pallas as pl
from jax.experimental.pallas import tpu as pltpu
from jax.experimental.pallas import tpu_sc as plsc

NC = 2
NS = 16
CH = 128
SB = 8
NBUF = 2
LANES = 16
DW = 128


def _mesh():
    return plsc.VectorSubcoreMesh(core_axis_name="c", subcore_axis_name="s")


def _sc_degree(dstp, n_pad):
    nsb = dstp.shape[2]
    rpw = n_pad // NS

    @functools.partial(
        pl.kernel,
        out_type=jax.ShapeDtypeStruct((NC, n_pad, DW), jnp.float32),
        mesh=_mesh(),
        scratch_types=[
            pltpu.VMEM((SB, CH), jnp.int32),
            pltpu.VMEM((CH, DW), jnp.float32),
            pltpu.VMEM((CH, DW), jnp.float32),
            pltpu.VMEM_SHARED((n_pad, DW), jnp.float32),
        ],
    )
    def deg_kernel(dst_hbm, out_hbm, dst_v, ones_v, zero_v, deg_sh):
        c = lax.axis_index("c")
        s = lax.axis_index("s")
        one16 = jnp.full((LANES,), 1.0, jnp.float32)
        z16 = jnp.zeros((LANES,), jnp.float32)

        @pl.loop(0, CH)
        def _(i):
            for u in range(DW // LANES):
                ones_v[i, pl.ds(u * LANES, LANES)] = one16
                zero_v[i, pl.ds(u * LANES, LANES)] = z16

        base = s * rpw
        @pl.loop(0, rpw // CH)
        def _(k):
            pltpu.sync_copy(zero_v, deg_sh.at[pl.ds(base + k * CH, CH), :])
        rem = rpw % CH
        if rem:
            pltpu.sync_copy(
                zero_v.at[pl.ds(0, rem), :],
                deg_sh.at[pl.ds(base + (rpw // CH) * CH, rem), :],
            )

        plsc.subcore_barrier()

        @pl.loop(0, nsb)
        def _(a):
            pltpu.sync_copy(dst_hbm.at[c, s, a], dst_v)
            for b in range(SB):
                pltpu.sync_copy(ones_v, deg_sh.at[dst_v.at[b]], add=True)

        plsc.subcore_barrier()
        pltpu.sync_copy(
            deg_sh.at[pl.ds(base, rpw), :], out_hbm.at[c, pl.ds(base, rpw), :]
        )

    return deg_kernel(dstp)


def _sc_scatter(g, srcp, dstp, n_pad):
    d = g.shape[1]
    nsb = srcp.shape[2]
    rpw = n_pad // NS

    @functools.partial(
        pl.kernel,
        out_type=jax.ShapeDtypeStruct((NC, n_pad, d), jnp.float32),
        mesh=_mesh(),
        scratch_types=[
            pltpu.VMEM((2, SB, CH), jnp.int32),
            pltpu.VMEM((2, SB, CH), jnp.int32),
            pltpu.VMEM((NBUF, CH, d), jnp.float32),
            pltpu.VMEM_SHARED((n_pad, d), jnp.float32),
            pltpu.SemaphoreType.DMA,
            pltpu.SemaphoreType.DMA,
            pltpu.SemaphoreType.DMA,
            pltpu.SemaphoreType.DMA,
            pltpu.SemaphoreType.DMA,
            pltpu.SemaphoreType.DMA,
        ],
    )
    def scat_kernel(g_hbm, src_hbm, dst_hbm, out_hbm,
                    si, di, ring, acc_sh,
                    sx0, sx1, sg0, sg1, ss0, ss1):
        c = lax.axis_index("c")
        s = lax.axis_index("s")
        z16 = jnp.zeros((LANES,), jnp.float32)
        isem = (sx0, sx1)
        gsem = (sg0, sg1)
        ssem = (ss0, ss1)

        def idx_start(sb, slot):
            pltpu.async_copy(src_hbm.at[c, s, sb], si.at[slot], isem[slot])
            pltpu.async_copy(dst_hbm.at[c, s, sb], di.at[slot], isem[slot])

        def idx_wait(sb, slot):
            pltpu.make_async_copy(
                src_hbm.at[c, s, sb], si.at[slot], isem[slot]).wait()
            pltpu.make_async_copy(
                dst_hbm.at[c, s, sb], di.at[slot], isem[slot]).wait()

        def gather_start(slot, j, buf):
            pltpu.async_copy(
                g_hbm.at[si.at[slot, j]], ring.at[buf], gsem[buf])

        def gather_wait(slot, j, buf):
            pltpu.make_async_copy(
                g_hbm.at[si.at[slot, j]], ring.at[buf], gsem[buf]).wait()

        def scat_start(slot, j, buf):
            pltpu.async_copy(
                ring.at[buf], acc_sh.at[di.at[slot, j]], ssem[buf],
                add=True)

        def scat_wait(slot, j, buf):
            pltpu.make_async_copy(
                ring.at[buf], acc_sh.at[di.at[slot, j]], ssem[buf]).wait()

        @pl.loop(0, CH)
        def _(i):
            for u in range(d // LANES):
                ring[0, i, pl.ds(u * LANES, LANES)] = z16

        base = s * rpw
        @pl.loop(0, rpw // CH)
        def _(k):
            pltpu.sync_copy(ring.at[0], acc_sh.at[pl.ds(base + k * CH, CH), :])
        rem = rpw % CH
        if rem:
            pltpu.sync_copy(
                ring.at[0, pl.ds(0, rem), :],
                acc_sh.at[pl.ds(base + (rpw // CH) * CH, rem), :],
            )

        idx_start(0, 0)
        idx_start(1, 1)
        idx_wait(0, 0)
        gather_start(0, 0, 0)
        plsc.subcore_barrier()

        @pl.loop(0, nsb, step=2)
        def _(sb):
            for p in range(2 * SB):
                slot, j = p // SB, p % SB
                nslot, nj = ((p + 1) // SB) % 2, (p + 1) % SB
                pslot, pj = ((p - 1) // SB) % 2, (p - 1) % SB
                buf = p % 2

                if p == 2:
                    @pl.when(sb > 0)
                    def _():
                        idx_start(sb + 1, 1)
                if p == SB - 2:
                    idx_wait(sb + 1, 1)
                if p == SB + 2:
                    @pl.when(sb + 2 < nsb)
                    def _():
                        idx_start(sb + 2, 0)
                if p == 2 * SB - 2:
                    @pl.when(sb + 2 < nsb)
                    def _():
                        idx_wait(sb + 2, 0)

                gather_wait(slot, j, buf)
                scat_start(slot, j, buf)
                if p < 2 * SB - 1:
                    if p == 0:
                        @pl.when(sb > 0)
                        def _():
                            scat_wait(1, SB - 1, 1 - buf)
                    else:
                        scat_wait(pslot, pj, 1 - buf)
                    gather_start(nslot, nj, 1 - buf)
                else:
                    @pl.when(sb + 2 < nsb)
                    def _():
                        scat_wait(pslot, pj, 1 - buf)
                        gather_start(0, 0, 1 - buf)

        scat_wait(1, SB - 2, 0)
        scat_wait(1, SB - 1, 1)

        plsc.subcore_barrier()
        pltpu.sync_copy(
            acc_sh.at[pl.ds(base, rpw), :], out_hbm.at[c, pl.ds(base, rpw), :]
        )

    return scat_kernel(g, srcp, dstp)


def _dis_block(da_ref):
    deg = da_ref[0, :, 0:1] + da_ref[1, :, 0:1] + 1.0
    return lax.rsqrt(deg)


def _dense1(x, w1, dega, r_blk):
    n, d = x.shape

    def body(x_ref, w_ref, da_ref, g_ref):
        dis = _dis_block(da_ref)
        p = jnp.dot(x_ref[...], w_ref[...], preferred_element_type=jnp.float32)
        g_ref[...] = p * dis

    return pl.pallas_call(
        body,
        grid=(n // r_blk,),
        in_specs=[
            pl.BlockSpec((r_blk, d), lambda r: (r, 0)),
            pl.BlockSpec((d, d), lambda r: (0, 0)),
            pl.BlockSpec((NC, r_blk, DW), lambda r: (0, r, 0)),
        ],
        out_specs=pl.BlockSpec((r_blk, d), lambda r: (r, 0)),
        out_shape=jax.ShapeDtypeStruct((n, d), jnp.float32),
    )(x, w1, dega)


def _dense2(acc, g1, dega, b1, w2, r_blk):
    n, d = g1.shape

    def body(a_ref, g_ref, da_ref, b_ref, w_ref, o_ref):
        dis = _dis_block(da_ref)
        h = dis * (a_ref[0] + a_ref[1] + g_ref[...]) + b_ref[...]
        h = jnp.maximum(h, 0.0)
        o_ref[...] = (
            jnp.dot(h, w_ref[...], preferred_element_type=jnp.float32) * dis
        )

    return pl.pallas_call(
        body,
        grid=(n // r_blk,),
        in_specs=[
            pl.BlockSpec((NC, r_blk, d), lambda r: (0, r, 0)),
            pl.BlockSpec((r_blk, d), lambda r: (r, 0)),
            pl.BlockSpec((NC, r_blk, DW), lambda r: (0, r, 0)),
            pl.BlockSpec((1, d), lambda r: (0, 0)),
            pl.BlockSpec((d, d), lambda r: (0, 0)),
        ],
        out_specs=pl.BlockSpec((r_blk, d), lambda r: (r, 0)),
        out_shape=jax.ShapeDtypeStruct((n, d), jnp.float32),
    )(acc, g1, dega, b1, w2)


def _dense3(acc, g2, dega, b2, r_blk):
    n, d = g2.shape

    def body(a_ref, g_ref, da_ref, b_ref, o_ref):
        dis = _dis_block(da_ref)
        t = dis * (a_ref[0] + a_ref[1] + g_ref[...]) + b_ref[...]
        m = jnp.max(t, axis=1, keepdims=True)
        u = t - m
        lse = jnp.log(jnp.sum(jnp.exp(u), axis=1, keepdims=True))
        o_ref[...] = u - lse

    return pl.pallas_call(
        body,
        grid=(n // r_blk,),
        in_specs=[
            pl.BlockSpec((NC, r_blk, d), lambda r: (0, r, 0)),
            pl.BlockSpec((r_blk, d), lambda r: (r, 0)),
            pl.BlockSpec((NC, r_blk, DW), lambda r: (0, r, 0)),
            pl.BlockSpec((1, d), lambda r: (0, 0)),
        ],
        out_specs=pl.BlockSpec((r_blk, d), lambda r: (r, 0)),
        out_shape=jax.ShapeDtypeStruct((n, d), jnp.float32),
    )(acc, g2, dega, b2)


def kernel(x, edge_index, W1, b1, W2, b2):
    n, d = x.shape
    e = edge_index.shape[1]

    per_round = NC * NS * CH
    nch = -(-e // per_round)
    nch = -(-nch // (2 * SB)) * (2 * SB)
    nsb = nch // SB
    e_pad = per_round * nch
    n_pad = (n // (NS * 8) + 1) * NS * 8

    src = edge_index[0]
    dst = edge_index[1]
    pad = e_pad - e
    srcp = jnp.concatenate(
        [src, jnp.zeros((pad,), jnp.int32)]
    ).reshape(NC, NS, nsb, SB, CH)
    dstp = jnp.concatenate(
        [dst, jnp.full((pad,), n, jnp.int32)]
    ).reshape(NC, NS, nsb, SB, CH)

    r_blk = 2000
    dega = _sc_degree(dstp, n_pad)
    g1 = _dense1(x, W1, dega, r_blk)
    acc1 = _sc_scatter(g1, srcp, dstp, n_pad)
    g2 = _dense2(acc1, g1, dega, b1.reshape(1, d), W2, r_blk)
    acc2 = _sc_scatter(g2, srcp, dstp, n_pad)
    return _dense3(acc2, g2, dega, b2.reshape(1, d), r_blk)


# speedup vs baseline: 1.1339x; 1.1339x over previous
import functools

import jax
import jax.numpy as jnp
from jax import lax
from jax.experimental import pallas as pl
from jax.experimental.pallas import tpu as pltpu
from jax.experimental.pallas import tpu_sc as plsc

NC = 2
NS = 16
CH = 64
SB = 8
NBUF = 4

LANES = 16
DW = 128


def _mesh():
    return plsc.VectorSubcoreMesh(core_axis_name="c", subcore_axis_name="s")


def _sc_degree(dstp, n_pad):
    nsb = dstp.shape[2]
    rpw = n_pad // NS

    @functools.partial(
        pl.kernel,
        out_type=jax.ShapeDtypeStruct((NC, n_pad, DW), jnp.float32),
        mesh=_mesh(),
        scratch_types=[
            pltpu.VMEM((SB, CH), jnp.int32),
            pltpu.VMEM((CH, DW), jnp.float32),
            pltpu.VMEM((CH, DW), jnp.float32),
            pltpu.VMEM_SHARED((n_pad, DW), jnp.float32),
        ],
    )
    def deg_kernel(dst_hbm, out_hbm, dst_v, ones_v, zero_v, deg_sh):
        c = lax.axis_index("c")
        s = lax.axis_index("s")
        one16 = jnp.full((LANES,), 1.0, jnp.float32)
        z16 = jnp.zeros((LANES,), jnp.float32)

        @pl.loop(0, CH)
        def _(i):
            for u in range(DW // LANES):
                ones_v[i, pl.ds(u * LANES, LANES)] = one16
                zero_v[i, pl.ds(u * LANES, LANES)] = z16

        base = s * rpw
        @pl.loop(0, rpw // CH)
        def _(k):
            pltpu.sync_copy(zero_v, deg_sh.at[pl.ds(base + k * CH, CH), :])
        rem = rpw % CH
        if rem:
            pltpu.sync_copy(
                zero_v.at[pl.ds(0, rem), :],
                deg_sh.at[pl.ds(base + (rpw // CH) * CH, rem), :],
            )

        plsc.subcore_barrier()

        @pl.loop(0, nsb)
        def _(a):
            pltpu.sync_copy(dst_hbm.at[c, s, a], dst_v)
            for b in range(SB):
                pltpu.sync_copy(ones_v, deg_sh.at[dst_v.at[b]], add=True)

        plsc.subcore_barrier()
        pltpu.sync_copy(
            deg_sh.at[pl.ds(base, rpw), :], out_hbm.at[c, pl.ds(base, rpw), :]
        )

    return deg_kernel(dstp)


def _sc_scatter(g, srcp, dstp, n_pad):
    d = g.shape[1]
    nsb = srcp.shape[2]
    rpw = n_pad // NS

    @functools.partial(
        pl.kernel,
        out_type=jax.ShapeDtypeStruct((NC, n_pad, d), jnp.float32),
        mesh=_mesh(),
        scratch_types=[
            pltpu.VMEM((2, SB, CH), jnp.int32),
            pltpu.VMEM((2, SB, CH), jnp.int32),
            pltpu.VMEM((NBUF, CH, d), jnp.float32),
            pltpu.VMEM_SHARED((n_pad, d), jnp.float32),
            pltpu.SemaphoreType.DMA,
            pltpu.SemaphoreType.DMA,
            pltpu.SemaphoreType.DMA,
            pltpu.SemaphoreType.DMA,
            pltpu.SemaphoreType.DMA,
            pltpu.SemaphoreType.DMA,
            pltpu.SemaphoreType.DMA,
            pltpu.SemaphoreType.DMA,
            pltpu.SemaphoreType.DMA,
            pltpu.SemaphoreType.DMA,
        ],
    )
    def scat_kernel(g_hbm, src_hbm, dst_hbm, out_hbm,
                    si, di, ring, acc_sh,
                    sx0, sx1, sg0, sg1, sg2, sg3, ss0, ss1, ss2, ss3):
        c = lax.axis_index("c")
        s = lax.axis_index("s")
        z16 = jnp.zeros((LANES,), jnp.float32)
        isem = (sx0, sx1)
        gsem = (sg0, sg1, sg2, sg3)
        ssem = (ss0, ss1, ss2, ss3)

        def idx_start(sb, slot):
            pltpu.async_copy(src_hbm.at[c, s, sb], si.at[slot], isem[slot])
            pltpu.async_copy(dst_hbm.at[c, s, sb], di.at[slot], isem[slot])

        def idx_wait(sb, slot):
            pltpu.make_async_copy(
                src_hbm.at[c, s, sb], si.at[slot], isem[slot]).wait()
            pltpu.make_async_copy(
                dst_hbm.at[c, s, sb], di.at[slot], isem[slot]).wait()

        def gather_start(slot, j, buf):
            pltpu.async_copy(
                g_hbm.at[si.at[slot, j]], ring.at[buf], gsem[buf])

        def gather_wait(slot, j, buf):
            pltpu.make_async_copy(
                g_hbm.at[si.at[slot, j]], ring.at[buf], gsem[buf]).wait()

        def scat_start(slot, j, buf):
            pltpu.async_copy(
                ring.at[buf], acc_sh.at[di.at[slot, j]], ssem[buf],
                add=True)

        def scat_wait(slot, j, buf):
            pltpu.make_async_copy(
                ring.at[buf], acc_sh.at[di.at[slot, j]], ssem[buf]).wait()

        @pl.loop(0, CH)
        def _(i):
            for u in range(d // LANES):
                ring[0, i, pl.ds(u * LANES, LANES)] = z16

        base = s * rpw
        @pl.loop(0, rpw // CH)
        def _(k):
            pltpu.sync_copy(ring.at[0], acc_sh.at[pl.ds(base + k * CH, CH), :])
        rem = rpw % CH
        if rem:
            pltpu.sync_copy(
                ring.at[0, pl.ds(0, rem), :],
                acc_sh.at[pl.ds(base + (rpw // CH) * CH, rem), :],
            )

        idx_start(0, 0)
        idx_start(1, 1)
        idx_wait(0, 0)
        gather_start(0, 0, 0)
        gather_start(0, 1, 1)
        plsc.subcore_barrier()

        @pl.loop(0, nsb, step=2)
        def _(sb):
            for p in range(2 * SB):
                slot, j = p // SB, p % SB
                nslot, nj = ((p + 2) // SB) % 2, (p + 2) % SB
                pslot, pj = ((p - 2) // SB) % 2, (p - 2) % SB
                buf, nbuf = p % NBUF, (p + 2) % NBUF

                if p == 2:
                    @pl.when(sb > 0)
                    def _():
                        idx_start(sb + 1, 1)
                if p == SB - 2:
                    idx_wait(sb + 1, 1)
                if p == SB + 2:
                    @pl.when(sb + 2 < nsb)
                    def _():
                        idx_start(sb + 2, 0)
                if p == 2 * SB - 3:
                    @pl.when(sb + 2 < nsb)
                    def _():
                        idx_wait(sb + 2, 0)

                if p < 2 * SB - 2:
                    if p < 2:
                        @pl.when(sb > 0)
                        def _():
                            scat_wait(1, SB - 2 + p, nbuf)
                    else:
                        scat_wait(pslot, pj, nbuf)
                    gather_start(nslot, nj, nbuf)
                else:
                    @pl.when(sb + 2 < nsb)
                    def _():
                        scat_wait(pslot, pj, nbuf)
                        gather_start(0, p - (2 * SB - 2), nbuf)
                gather_wait(slot, j, buf)
                scat_start(slot, j, buf)

        for p in range(2 * SB - NBUF, 2 * SB):
            scat_wait(p // SB, p % SB, p % NBUF)

        plsc.subcore_barrier()
        pltpu.sync_copy(
            acc_sh.at[pl.ds(base, rpw), :], out_hbm.at[c, pl.ds(base, rpw), :]
        )

    return scat_kernel(g, srcp, dstp)


def _dis_block(da_ref):
    deg = da_ref[0, :, 0:1] + da_ref[1, :, 0:1] + 1.0
    return lax.rsqrt(deg)


def _dense1(x, w1, dega, r_blk):
    n, d = x.shape

    def body(x_ref, w_ref, da_ref, g_ref):
        dis = _dis_block(da_ref)
        p = jnp.dot(x_ref[...], w_ref[...], preferred_element_type=jnp.float32)
        g_ref[...] = p * dis

    return pl.pallas_call(
        body,
        grid=(n // r_blk,),
        in_specs=[
            pl.BlockSpec((r_blk, d), lambda r: (r, 0)),
            pl.BlockSpec((d, d), lambda r: (0, 0)),
            pl.BlockSpec((NC, r_blk, DW), lambda r: (0, r, 0)),
        ],
        out_specs=pl.BlockSpec((r_blk, d), lambda r: (r, 0)),
        out_shape=jax.ShapeDtypeStruct((n, d), jnp.float32),
    )(x, w1, dega)


def _dense2(acc, g1, dega, b1, w2, r_blk):
    n, d = g1.shape

    def body(a_ref, g_ref, da_ref, b_ref, w_ref, o_ref):
        dis = _dis_block(da_ref)
        h = dis * (a_ref[0] + a_ref[1] + g_ref[...]) + b_ref[...]
        h = jnp.maximum(h, 0.0)
        o_ref[...] = (
            jnp.dot(h, w_ref[...], preferred_element_type=jnp.float32) * dis
        )

    return pl.pallas_call(
        body,
        grid=(n // r_blk,),
        in_specs=[
            pl.BlockSpec((NC, r_blk, d), lambda r: (0, r, 0)),
            pl.BlockSpec((r_blk, d), lambda r: (r, 0)),
            pl.BlockSpec((NC, r_blk, DW), lambda r: (0, r, 0)),
            pl.BlockSpec((1, d), lambda r: (0, 0)),
            pl.BlockSpec((d, d), lambda r: (0, 0)),
        ],
        out_specs=pl.BlockSpec((r_blk, d), lambda r: (r, 0)),
        out_shape=jax.ShapeDtypeStruct((n, d), jnp.float32),
    )(acc, g1, dega, b1, w2)


def _dense3(acc, g2, dega, b2, r_blk):
    n, d = g2.shape

    def body(a_ref, g_ref, da_ref, b_ref, o_ref):
        dis = _dis_block(da_ref)
        t = dis * (a_ref[0] + a_ref[1] + g_ref[...]) + b_ref[...]
        m = jnp.max(t, axis=1, keepdims=True)
        u = t - m
        lse = jnp.log(jnp.sum(jnp.exp(u), axis=1, keepdims=True))
        o_ref[...] = u - lse

    return pl.pallas_call(
        body,
        grid=(n // r_blk,),
        in_specs=[
            pl.BlockSpec((NC, r_blk, d), lambda r: (0, r, 0)),
            pl.BlockSpec((r_blk, d), lambda r: (r, 0)),
            pl.BlockSpec((NC, r_blk, DW), lambda r: (0, r, 0)),
            pl.BlockSpec((1, d), lambda r: (0, 0)),
        ],
        out_specs=pl.BlockSpec((r_blk, d), lambda r: (r, 0)),
        out_shape=jax.ShapeDtypeStruct((n, d), jnp.float32),
    )(acc, g2, dega, b2)


def kernel(x, edge_index, W1, b1, W2, b2):
    n, d = x.shape
    e = edge_index.shape[1]

    per_round = NC * NS * CH
    nch = -(-e // per_round)
    nch = -(-nch // (2 * SB)) * (2 * SB)
    nsb = nch // SB
    e_pad = per_round * nch
    n_pad = (n // (NS * 8) + 1) * NS * 8

    src = edge_index[0]
    dst = edge_index[1]
    pad = e_pad - e
    srcp = jnp.concatenate(
        [src, jnp.zeros((pad,), jnp.int32)]
    ).reshape(NC, NS, nsb, SB, CH)
    dstp = jnp.concatenate(
        [dst, jnp.full((pad,), n, jnp.int32)]
    ).reshape(NC, NS, nsb, SB, CH)

    r_blk = 2000
    dega = _sc_degree(dstp, n_pad)
    g1 = _dense1(x, W1, dega, r_blk)
    acc1 = _sc_scatter(g1, srcp, dstp, n_pad)
    g2 = _dense2(acc1, g1, dega, b1.reshape(1, d), W2, r_blk)
    acc2 = _sc_scatter(g2, srcp, dstp, n_pad)
    return _dense3(acc2, g2, dega, b2.reshape(1, d), r_blk)


# speedup vs baseline: 3.3175x; 2.9256x over previous
import functools

import jax
import jax.numpy as jnp
from jax import lax
from jax.experimental import pallas as pl
from jax.experimental.pallas import tpu as pltpu
from jax.experimental.pallas import tpu_sc as plsc

NC = 2
NS = 16
CH = 64
SB = 8
NBUF = 4

LANES = 16
DW = 128


def _mesh():
    return plsc.VectorSubcoreMesh(core_axis_name="c", subcore_axis_name="s")


def _sc_degree(dstp, n_pad):
    nsb = dstp.shape[2]
    rpw = n_pad // NS

    @functools.partial(
        pl.kernel,
        out_type=jax.ShapeDtypeStruct((NC, n_pad, DW), jnp.float32),
        mesh=_mesh(),
        scratch_types=[
            pltpu.VMEM((SB, CH), jnp.int32),
            pltpu.VMEM((CH, DW), jnp.float32),
            pltpu.VMEM((CH, DW), jnp.float32),
            pltpu.VMEM_SHARED((n_pad, DW), jnp.float32),
        ],
    )
    def deg_kernel(dst_hbm, out_hbm, dst_v, ones_v, zero_v, deg_sh):
        c = lax.axis_index("c")
        s = lax.axis_index("s")
        one16 = jnp.full((LANES,), 1.0, jnp.float32)
        z16 = jnp.zeros((LANES,), jnp.float32)

        @pl.loop(0, CH)
        def _(i):
            for u in range(DW // LANES):
                ones_v[i, pl.ds(u * LANES, LANES)] = one16
                zero_v[i, pl.ds(u * LANES, LANES)] = z16

        base = s * rpw
        @pl.loop(0, rpw // CH)
        def _(k):
            pltpu.sync_copy(zero_v, deg_sh.at[pl.ds(base + k * CH, CH), :])
        rem = rpw % CH
        if rem:
            pltpu.sync_copy(
                zero_v.at[pl.ds(0, rem), :],
                deg_sh.at[pl.ds(base + (rpw // CH) * CH, rem), :],
            )

        plsc.subcore_barrier()

        @pl.loop(0, nsb)
        def _(a):
            pltpu.sync_copy(dst_hbm.at[c, s, a], dst_v)
            for b in range(SB):
                pltpu.sync_copy(ones_v, deg_sh.at[dst_v.at[b]], add=True)

        plsc.subcore_barrier()
        pltpu.sync_copy(
            deg_sh.at[pl.ds(base, rpw), :], out_hbm.at[c, pl.ds(base, rpw), :]
        )

    return deg_kernel(dstp)


def _sc_scatter(g, srcp, dstp, n_pad):
    d = g.shape[1]
    nsb = srcp.shape[2]
    rpw = n_pad // NS

    @functools.partial(
        pl.kernel,
        out_type=jax.ShapeDtypeStruct((NC, n_pad, d), jnp.float32),
        mesh=_mesh(),
        scratch_types=[
            pltpu.VMEM((2, SB, CH), jnp.int32),
            pltpu.VMEM((2, SB, CH), jnp.int32),
            pltpu.VMEM((NBUF, CH, d), jnp.float32),
            pltpu.VMEM_SHARED((n_pad, d), jnp.float32),
            pltpu.SemaphoreType.DMA,
            pltpu.SemaphoreType.DMA,
            pltpu.SemaphoreType.DMA,
            pltpu.SemaphoreType.DMA,
            pltpu.SemaphoreType.DMA,
            pltpu.SemaphoreType.DMA,
            pltpu.SemaphoreType.DMA,
            pltpu.SemaphoreType.DMA,
            pltpu.SemaphoreType.DMA,
            pltpu.SemaphoreType.DMA,
        ],
    )
    def scat_kernel(g_hbm, src_hbm, dst_hbm, out_hbm,
                    si, di, ring, acc_sh,
                    sx0, sx1, sg0, sg1, sg2, sg3, ss0, ss1, ss2, ss3):
        c = lax.axis_index("c")
        s = lax.axis_index("s")
        z16 = jnp.zeros((LANES,), jnp.float32)
        isem = (sx0, sx1)
        gsem = (sg0, sg1, sg2, sg3)
        ssem = (ss0, ss1, ss2, ss3)

        def idx_start(sb, slot):
            pltpu.async_copy(src_hbm.at[c, s, sb], si.at[slot], isem[slot])
            pltpu.async_copy(dst_hbm.at[c, s, sb], di.at[slot], isem[slot])

        def idx_wait(sb, slot):
            pltpu.make_async_copy(
                src_hbm.at[c, s, sb], si.at[slot], isem[slot]).wait()
            pltpu.make_async_copy(
                dst_hbm.at[c, s, sb], di.at[slot], isem[slot]).wait()

        def gather_start(slot, j, buf):
            pltpu.async_copy(
                g_hbm.at[si.at[slot, j]], ring.at[buf], gsem[buf])

        def gather_wait(slot, j, buf):
            pltpu.make_async_copy(
                g_hbm.at[si.at[slot, j]], ring.at[buf], gsem[buf]).wait()

        def scat_start(slot, j, buf):
            pltpu.async_copy(
                ring.at[buf], acc_sh.at[di.at[slot, j]], ssem[buf],
                add=True)

        def scat_wait(slot, j, buf):
            pltpu.make_async_copy(
                ring.at[buf], acc_sh.at[di.at[slot, j]], ssem[buf]).wait()

        @pl.loop(0, CH)
        def _(i):
            for u in range(d // LANES):
                ring[0, i, pl.ds(u * LANES, LANES)] = z16

        base = s * rpw
        @pl.loop(0, rpw // CH)
        def _(k):
            pltpu.sync_copy(ring.at[0], acc_sh.at[pl.ds(base + k * CH, CH), :])
        rem = rpw % CH
        if rem:
            pltpu.sync_copy(
                ring.at[0, pl.ds(0, rem), :],
                acc_sh.at[pl.ds(base + (rpw // CH) * CH, rem), :],
            )

        idx_start(0, 0)
        idx_start(1, 1)
        idx_wait(0, 0)
        gather_start(0, 0, 0)
        gather_start(0, 1, 1)
        plsc.subcore_barrier()

        @pl.loop(0, nsb, step=2)
        def _(sb):
            for p in range(2 * SB):
                slot, j = p // SB, p % SB
                nslot, nj = ((p + 2) // SB) % 2, (p + 2) % SB
                pslot, pj = ((p - 2) // SB) % 2, (p - 2) % SB
                buf, nbuf = p % NBUF, (p + 2) % NBUF

                if p == 2:
                    @pl.when(sb > 0)
                    def _():
                        idx_start(sb + 1, 1)
                if p == SB - 2:
                    idx_wait(sb + 1, 1)
                if p == SB + 2:
                    @pl.when(sb + 2 < nsb)
                    def _():
                        idx_start(sb + 2, 0)
                if p == 2 * SB - 3:
                    @pl.when(sb + 2 < nsb)
                    def _():
                        idx_wait(sb + 2, 0)

                if p < 2 * SB - 2:
                    if p < 2:
                        @pl.when(sb > 0)
                        def _():
                            scat_wait(1, SB - 2 + p, nbuf)
                    else:
                        scat_wait(pslot, pj, nbuf)
                    gather_start(nslot, nj, nbuf)
                else:
                    @pl.when(sb + 2 < nsb)
                    def _():
                        scat_wait(pslot, pj, nbuf)
                        gather_start(0, p - (2 * SB - 2), nbuf)
                gather_wait(slot, j, buf)
                scat_start(slot, j, buf)

        for p in range(2 * SB - NBUF, 2 * SB):
            scat_wait(p // SB, p % SB, p % NBUF)

        plsc.subcore_barrier()
        pltpu.sync_copy(
            acc_sh.at[pl.ds(base, rpw), :], out_hbm.at[c, pl.ds(base, rpw), :]
        )

    return scat_kernel(g, srcp, dstp)


def _dis_block(da_ref):
    deg = da_ref[0, :, 0:1] + da_ref[1, :, 0:1] + 1.0
    return lax.rsqrt(deg)


def _dense1(x, w1, dega, r_blk):
    n, d = x.shape

    def body(x_ref, w_ref, da_ref, g_ref):
        dis = _dis_block(da_ref)
        p = jnp.dot(x_ref[...], w_ref[...], preferred_element_type=jnp.float32)
        g_ref[...] = p * dis

    return pl.pallas_call(
        body,
        grid=(n // r_blk,),
        in_specs=[
            pl.BlockSpec((r_blk, d), lambda r: (r, 0)),
            pl.BlockSpec((d, d), lambda r: (0, 0)),
            pl.BlockSpec((NC, r_blk, DW), lambda r: (0, r, 0)),
        ],
        out_specs=pl.BlockSpec((r_blk, d), lambda r: (r, 0)),
        out_shape=jax.ShapeDtypeStruct((n, d), jnp.float32),
    )(x, w1, dega)


def _dense2(acc, g1, dega, b1, w2, r_blk):
    n, d = g1.shape

    def body(a_ref, g_ref, da_ref, b_ref, w_ref, o_ref):
        dis = _dis_block(da_ref)
        h = dis * (a_ref[0] + a_ref[1] + g_ref[...]) + b_ref[...]
        h = jnp.maximum(h, 0.0)
        o_ref[...] = (
            jnp.dot(h, w_ref[...], preferred_element_type=jnp.float32) * dis
        )

    return pl.pallas_call(
        body,
        grid=(n // r_blk,),
        in_specs=[
            pl.BlockSpec((NC, r_blk, d), lambda r: (0, r, 0)),
            pl.BlockSpec((r_blk, d), lambda r: (r, 0)),
            pl.BlockSpec((NC, r_blk, DW), lambda r: (0, r, 0)),
            pl.BlockSpec((1, d), lambda r: (0, 0)),
            pl.BlockSpec((d, d), lambda r: (0, 0)),
        ],
        out_specs=pl.BlockSpec((r_blk, d), lambda r: (r, 0)),
        out_shape=jax.ShapeDtypeStruct((n, d), jnp.float32),
    )(acc, g1, dega, b1, w2)


def _dense3(acc, g2, dega, b2, r_blk):
    n, d = g2.shape

    def body(a_ref, g_ref, da_ref, b_ref, o_ref):
        dis = _dis_block(da_ref)
        t = dis * (a_ref[0] + a_ref[1] + g_ref[...]) + b_ref[...]
        m = jnp.max(t, axis=1, keepdims=True)
        u = t - m
        lse = jnp.log(jnp.sum(jnp.exp(u), axis=1, keepdims=True))
        o_ref[...] = u - lse

    return pl.pallas_call(
        body,
        grid=(n // r_blk,),
        in_specs=[
            pl.BlockSpec((NC, r_blk, d), lambda r: (0, r, 0)),
            pl.BlockSpec((r_blk, d), lambda r: (r, 0)),
            pl.BlockSpec((NC, r_blk, DW), lambda r: (0, r, 0)),
            pl.BlockSpec((1, d), lambda r: (0, 0)),
        ],
        out_specs=pl.BlockSpec((r_blk, d), lambda r: (r, 0)),
        out_shape=jax.ShapeDtypeStruct((n, d), jnp.float32),
    )(acc, g2, dega, b2)


def kernel(x, edge_index, W1, b1, W2, b2):
    n, d = x.shape
    e = edge_index.shape[1]

    per_round = NC * NS * CH
    nch = -(-e // per_round)
    nch = -(-nch // (2 * SB)) * (2 * SB)
    nsb = nch // SB
    e_pad = per_round * nch
    n_pad = (n // (NS * 8) + 1) * NS * 8

    src = edge_index[0]
    dst = edge_index[1]
    pad = e_pad - e
    pad_i = jnp.arange(pad, dtype=jnp.int32)
    srcp = jnp.concatenate(
        [src, pad_i % n]
    ).reshape(NC, NS, nsb, SB, CH)
    dstp = jnp.concatenate(
        [dst, n + pad_i % (n_pad - n)]
    ).reshape(NC, NS, nsb, SB, CH)

    r_blk = 2000
    dega = _sc_degree(dstp, n_pad)
    g1 = _dense1(x, W1, dega, r_blk)
    acc1 = _sc_scatter(g1, srcp, dstp, n_pad)
    g2 = _dense2(acc1, g1, dega, b1.reshape(1, d), W2, r_blk)
    acc2 = _sc_scatter(g2, srcp, dstp, n_pad)
    return _dense3(acc2, g2, dega, b2.reshape(1, d), r_blk)


# speedup vs baseline: 3.4471x; 1.0391x over previous
import functools

import jax
import jax.numpy as jnp
from jax import lax
from jax.experimental import pallas as pl
from jax.experimental.pallas import tpu as pltpu
from jax.experimental.pallas import tpu_sc as plsc

NC = 2
NS = 16
CH = 64
SB = 8
NBUF = 4

LANES = 16
DW = 128


def _mesh():
    return plsc.VectorSubcoreMesh(core_axis_name="c", subcore_axis_name="s")


def _sc_degree(dstp, n_pad):
    nsb = dstp.shape[2]
    rpw = n_pad // NS

    @functools.partial(
        pl.kernel,
        out_type=jax.ShapeDtypeStruct((NC, n_pad, DW), jnp.float32),
        mesh=_mesh(),
        scratch_types=[
            pltpu.VMEM((2, SB, CH), jnp.int32),
            pltpu.VMEM((CH, DW), jnp.float32),
            pltpu.VMEM((CH, DW), jnp.float32),
            pltpu.VMEM_SHARED((n_pad, DW), jnp.float32),
            pltpu.SemaphoreType.DMA,
            pltpu.SemaphoreType.DMA,
            pltpu.SemaphoreType.DMA,
            pltpu.SemaphoreType.DMA,
            pltpu.SemaphoreType.DMA,
            pltpu.SemaphoreType.DMA,
        ],
    )
    def deg_kernel(dst_hbm, out_hbm, di, ones_v, zero_v, deg_sh,
                   sx0, sx1, ss0, ss1, ss2, ss3):
        c = lax.axis_index("c")
        s = lax.axis_index("s")
        one16 = jnp.full((LANES,), 1.0, jnp.float32)
        z16 = jnp.zeros((LANES,), jnp.float32)
        isem = (sx0, sx1)
        ssem = (ss0, ss1, ss2, ss3)

        def idx_start(sb, slot):
            pltpu.async_copy(dst_hbm.at[c, s, sb], di.at[slot], isem[slot])

        def idx_wait(sb, slot):
            pltpu.make_async_copy(
                dst_hbm.at[c, s, sb], di.at[slot], isem[slot]).wait()

        def scat_start(slot, b, q):
            pltpu.async_copy(
                ones_v, deg_sh.at[di.at[slot, b]], ssem[q], add=True)

        def scat_wait(slot, b, q):
            pltpu.make_async_copy(
                ones_v, deg_sh.at[di.at[slot, b]], ssem[q]).wait()

        @pl.loop(0, CH)
        def _(i):
            for u in range(DW // LANES):
                ones_v[i, pl.ds(u * LANES, LANES)] = one16
                zero_v[i, pl.ds(u * LANES, LANES)] = z16

        base = s * rpw
        @pl.loop(0, rpw // CH)
        def _(k):
            pltpu.sync_copy(zero_v, deg_sh.at[pl.ds(base + k * CH, CH), :])
        rem = rpw % CH
        if rem:
            pltpu.sync_copy(
                zero_v.at[pl.ds(0, rem), :],
                deg_sh.at[pl.ds(base + (rpw // CH) * CH, rem), :],
            )

        idx_start(0, 0)
        idx_start(1, 1)
        idx_wait(0, 0)
        plsc.subcore_barrier()

        @pl.loop(0, nsb, step=2)
        def _(a):
            for p in range(2 * SB):
                slot, b = p // SB, p % SB
                q = p % 4
                if p < 4:
                    @pl.when(a > 0)
                    def _():
                        scat_wait(1, SB - 4 + p, q)
                else:
                    scat_wait((p - 4) // SB, (p - 4) % SB, q)
                if p == 3:
                    @pl.when(a > 0)
                    def _():
                        idx_start(a + 1, 1)
                if p == SB - 2:
                    idx_wait(a + 1, 1)
                if p == SB + 3:
                    @pl.when(a + 2 < nsb)
                    def _():
                        idx_start(a + 2, 0)
                if p == 2 * SB - 1:
                    @pl.when(a + 2 < nsb)
                    def _():
                        idx_wait(a + 2, 0)
                scat_start(slot, b, q)

        for p in range(2 * SB - 4, 2 * SB):
            scat_wait(p // SB, p % SB, p % 4)

        plsc.subcore_barrier()
        pltpu.sync_copy(
            deg_sh.at[pl.ds(base, rpw), :], out_hbm.at[c, pl.ds(base, rpw), :]
        )

    return deg_kernel(dstp)


def _sc_scatter(g, srcp, dstp, n_pad):
    d = g.shape[1]
    nsb = srcp.shape[2]
    rpw = n_pad // NS

    @functools.partial(
        pl.kernel,
        out_type=jax.ShapeDtypeStruct((NC, n_pad, d), jnp.float32),
        mesh=_mesh(),
        scratch_types=[
            pltpu.VMEM((2, SB, CH), jnp.int32),
            pltpu.VMEM((2, SB, CH), jnp.int32),
            pltpu.VMEM((NBUF, CH, d), jnp.float32),
            pltpu.VMEM_SHARED((n_pad, d), jnp.float32),
            pltpu.SemaphoreType.DMA,
            pltpu.SemaphoreType.DMA,
            pltpu.SemaphoreType.DMA,
            pltpu.SemaphoreType.DMA,
            pltpu.SemaphoreType.DMA,
            pltpu.SemaphoreType.DMA,
            pltpu.SemaphoreType.DMA,
            pltpu.SemaphoreType.DMA,
            pltpu.SemaphoreType.DMA,
            pltpu.SemaphoreType.DMA,
        ],
    )
    def scat_kernel(g_hbm, src_hbm, dst_hbm, out_hbm,
                    si, di, ring, acc_sh,
                    sx0, sx1, sg0, sg1, sg2, sg3, ss0, ss1, ss2, ss3):
        c = lax.axis_index("c")
        s = lax.axis_index("s")
        z16 = jnp.zeros((LANES,), jnp.float32)
        isem = (sx0, sx1)
        gsem = (sg0, sg1, sg2, sg3)
        ssem = (ss0, ss1, ss2, ss3)

        def idx_start(sb, slot):
            pltpu.async_copy(src_hbm.at[c, s, sb], si.at[slot], isem[slot])
            pltpu.async_copy(dst_hbm.at[c, s, sb], di.at[slot], isem[slot])

        def idx_wait(sb, slot):
            pltpu.make_async_copy(
                src_hbm.at[c, s, sb], si.at[slot], isem[slot]).wait()
            pltpu.make_async_copy(
                dst_hbm.at[c, s, sb], di.at[slot], isem[slot]).wait()

        def gather_start(slot, j, buf):
            pltpu.async_copy(
                g_hbm.at[si.at[slot, j]], ring.at[buf], gsem[buf])

        def gather_wait(slot, j, buf):
            pltpu.make_async_copy(
                g_hbm.at[si.at[slot, j]], ring.at[buf], gsem[buf]).wait()

        def scat_start(slot, j, buf):
            pltpu.async_copy(
                ring.at[buf], acc_sh.at[di.at[slot, j]], ssem[buf],
                add=True)

        def scat_wait(slot, j, buf):
            pltpu.make_async_copy(
                ring.at[buf], acc_sh.at[di.at[slot, j]], ssem[buf]).wait()

        @pl.loop(0, CH)
        def _(i):
            for u in range(d // LANES):
                ring[0, i, pl.ds(u * LANES, LANES)] = z16

        base = s * rpw
        @pl.loop(0, rpw // CH)
        def _(k):
            pltpu.sync_copy(ring.at[0], acc_sh.at[pl.ds(base + k * CH, CH), :])
        rem = rpw % CH
        if rem:
            pltpu.sync_copy(
                ring.at[0, pl.ds(0, rem), :],
                acc_sh.at[pl.ds(base + (rpw // CH) * CH, rem), :],
            )

        idx_start(0, 0)
        idx_start(1, 1)
        idx_wait(0, 0)
        gather_start(0, 0, 0)
        gather_start(0, 1, 1)
        plsc.subcore_barrier()

        @pl.loop(0, nsb, step=2)
        def _(sb):
            for p in range(2 * SB):
                slot, j = p // SB, p % SB
                nslot, nj = ((p + 2) // SB) % 2, (p + 2) % SB
                pslot, pj = ((p - 2) // SB) % 2, (p - 2) % SB
                buf, nbuf = p % NBUF, (p + 2) % NBUF

                if p == 2:
                    @pl.when(sb > 0)
                    def _():
                        idx_start(sb + 1, 1)
                if p == SB - 2:
                    idx_wait(sb + 1, 1)
                if p == SB + 2:
                    @pl.when(sb + 2 < nsb)
                    def _():
                        idx_start(sb + 2, 0)
                if p == 2 * SB - 3:
                    @pl.when(sb + 2 < nsb)
                    def _():
                        idx_wait(sb + 2, 0)

                if p < 2 * SB - 2:
                    if p < 2:
                        @pl.when(sb > 0)
                        def _():
                            scat_wait(1, SB - 2 + p, nbuf)
                    else:
                        scat_wait(pslot, pj, nbuf)
                    gather_start(nslot, nj, nbuf)
                else:
                    @pl.when(sb + 2 < nsb)
                    def _():
                        scat_wait(pslot, pj, nbuf)
                        gather_start(0, p - (2 * SB - 2), nbuf)
                gather_wait(slot, j, buf)
                scat_start(slot, j, buf)

        for p in range(2 * SB - NBUF, 2 * SB):
            scat_wait(p // SB, p % SB, p % NBUF)

        plsc.subcore_barrier()
        pltpu.sync_copy(
            acc_sh.at[pl.ds(base, rpw), :], out_hbm.at[c, pl.ds(base, rpw), :]
        )

    return scat_kernel(g, srcp, dstp)


def _dis_block(da_ref):
    deg = da_ref[0, :, 0:1] + da_ref[1, :, 0:1] + 1.0
    return lax.rsqrt(deg)


def _dense1(x, w1, dega, r_blk):
    n, d = x.shape

    def body(x_ref, w_ref, da_ref, g_ref):
        dis = _dis_block(da_ref)
        p = jnp.dot(x_ref[...], w_ref[...], preferred_element_type=jnp.float32)
        g_ref[...] = p * dis

    return pl.pallas_call(
        body,
        grid=(n // r_blk,),
        in_specs=[
            pl.BlockSpec((r_blk, d), lambda r: (r, 0)),
            pl.BlockSpec((d, d), lambda r: (0, 0)),
            pl.BlockSpec((NC, r_blk, DW), lambda r: (0, r, 0)),
        ],
        out_specs=pl.BlockSpec((r_blk, d), lambda r: (r, 0)),
        out_shape=jax.ShapeDtypeStruct((n, d), jnp.float32),
    )(x, w1, dega)


def _dense2(acc, g1, dega, b1, w2, r_blk):
    n, d = g1.shape

    def body(a_ref, g_ref, da_ref, b_ref, w_ref, o_ref):
        dis = _dis_block(da_ref)
        h = dis * (a_ref[0] + a_ref[1] + g_ref[...]) + b_ref[...]
        h = jnp.maximum(h, 0.0)
        o_ref[...] = (
            jnp.dot(h, w_ref[...], preferred_element_type=jnp.float32) * dis
        )

    return pl.pallas_call(
        body,
        grid=(n // r_blk,),
        in_specs=[
            pl.BlockSpec((NC, r_blk, d), lambda r: (0, r, 0)),
            pl.BlockSpec((r_blk, d), lambda r: (r, 0)),
            pl.BlockSpec((NC, r_blk, DW), lambda r: (0, r, 0)),
            pl.BlockSpec((1, d), lambda r: (0, 0)),
            pl.BlockSpec((d, d), lambda r: (0, 0)),
        ],
        out_specs=pl.BlockSpec((r_blk, d), lambda r: (r, 0)),
        out_shape=jax.ShapeDtypeStruct((n, d), jnp.float32),
    )(acc, g1, dega, b1, w2)


def _dense3(acc, g2, dega, b2, r_blk):
    n, d = g2.shape

    def body(a_ref, g_ref, da_ref, b_ref, o_ref):
        dis = _dis_block(da_ref)
        t = dis * (a_ref[0] + a_ref[1] + g_ref[...]) + b_ref[...]
        m = jnp.max(t, axis=1, keepdims=True)
        u = t - m
        lse = jnp.log(jnp.sum(jnp.exp(u), axis=1, keepdims=True))
        o_ref[...] = u - lse

    return pl.pallas_call(
        body,
        grid=(n // r_blk,),
        in_specs=[
            pl.BlockSpec((NC, r_blk, d), lambda r: (0, r, 0)),
            pl.BlockSpec((r_blk, d), lambda r: (r, 0)),
            pl.BlockSpec((NC, r_blk, DW), lambda r: (0, r, 0)),
            pl.BlockSpec((1, d), lambda r: (0, 0)),
        ],
        out_specs=pl.BlockSpec((r_blk, d), lambda r: (r, 0)),
        out_shape=jax.ShapeDtypeStruct((n, d), jnp.float32),
    )(acc, g2, dega, b2)


def kernel(x, edge_index, W1, b1, W2, b2):
    n, d = x.shape
    e = edge_index.shape[1]

    per_round = NC * NS * CH
    nch = -(-e // per_round)
    nch = -(-nch // (2 * SB)) * (2 * SB)
    nsb = nch // SB
    e_pad = per_round * nch
    n_pad = (n // (NS * 8) + 1) * NS * 8

    src = edge_index[0]
    dst = edge_index[1]
    pad = e_pad - e
    pad_i = jnp.arange(pad, dtype=jnp.int32)
    srcp = jnp.concatenate(
        [src, pad_i % n]
    ).reshape(NC, NS, nsb, SB, CH)
    dstp = jnp.concatenate(
        [dst, n + pad_i % (n_pad - n)]
    ).reshape(NC, NS, nsb, SB, CH)

    r_blk = 2000
    dega = _sc_degree(dstp, n_pad)
    g1 = _dense1(x, W1, dega, r_blk)
    acc1 = _sc_scatter(g1, srcp, dstp, n_pad)
    g2 = _dense2(acc1, g1, dega, b1.reshape(1, d), W2, r_blk)
    acc2 = _sc_scatter(g2, srcp, dstp, n_pad)
    return _dense3(acc2, g2, dega, b2.reshape(1, d), r_blk)
